# Initial kernel scaffold; baseline (speedup 1.0000x reference)
import functools

import jax
import jax.numpy as jnp
from jax import lax
from jax.experimental import pallas as pl
from jax.experimental.pallas import tpu as pltpu
from jax.experimental.pallas import tpu_sc as plsc

N_NODES = 10000
D_FEAT = 128
N_EDGES = 320000

NC = 2
NS = 16
K = 128
S = (N_EDGES + NC * NS * K - 1) // (NC * NS * K)
E_PAD = NC * NS * S * K
ROWS_PER_SUB = N_NODES // NS


def _sc_body(x_hbm, src_hbm, dst_hbm, w_hbm, part_hbm,
             src_v, dst_v, w_v, rows_v, acc, sem):
    c = lax.axis_index("c")
    s = lax.axis_index("s")

    pltpu.sync_copy(src_hbm.at[c, s], src_v)
    pltpu.sync_copy(dst_hbm.at[c, s], dst_v)
    pltpu.sync_copy(w_hbm.at[c, s], w_v)

    zero16 = jnp.zeros((16,), jnp.float32)

    def _zrow(i, _):
        for r in range(D_FEAT // 16):
            rows_v[i, pl.ds(r * 16, 16)] = zero16
        return 0

    lax.fori_loop(0, K, _zrow, 0)
    row0 = s * ROWS_PER_SUB
    off = 0
    while off < ROWS_PER_SUB:
        n = min(K, ROWS_PER_SUB - off)
        pltpu.sync_copy(rows_v.at[pl.ds(0, n)], acc.at[pl.ds(row0 + off, n)])
        off += n
    plsc.subcore_barrier()

    def _step(j, _):
        pltpu.async_copy(x_hbm.at[src_v.at[j]], rows_v, sem).wait()

        def _scale(k, _):
            wv = w_v[j, k]
            for r in range(D_FEAT // 16):
                sl = pl.ds(r * 16, 16)
                rows_v[k, sl] = rows_v[k, sl] * wv
            return 0

        lax.fori_loop(0, K, _scale, 0)
        pltpu.sync_copy(rows_v, acc.at[dst_v.at[j]], add=True)
        return 0

    lax.fori_loop(0, S, _step, 0)
    plsc.subcore_barrier()

    pltpu.sync_copy(acc.at[pl.ds(row0, ROWS_PER_SUB)],
                    part_hbm.at[c, pl.ds(row0, ROWS_PER_SUB)])


_sc_spmm = functools.partial(
    pl.kernel,
    _sc_body,
    out_type=jax.ShapeDtypeStruct((NC, N_NODES, D_FEAT), jnp.float32),
    mesh=plsc.VectorSubcoreMesh(core_axis_name="c", subcore_axis_name="s"),
    scratch_types=[
        pltpu.VMEM((S, K), jnp.int32),
        pltpu.VMEM((S, K), jnp.int32),
        pltpu.VMEM((S, K), jnp.float32),
        pltpu.VMEM((K, D_FEAT), jnp.float32),
        pltpu.VMEM_SHARED((N_NODES, D_FEAT), jnp.float32),
        pltpu.SemaphoreType.DMA,
    ],
)()


def _add_body(a_ref, b_ref, o_ref):
    o_ref[...] = a_ref[...] + b_ref[...]


_BLK = 1000
_tc_add = pl.pallas_call(
    _add_body,
    grid=(N_NODES // _BLK,),
    in_specs=[pl.BlockSpec((_BLK, D_FEAT), lambda i: (i, 0)),
              pl.BlockSpec((_BLK, D_FEAT), lambda i: (i, 0))],
    out_specs=pl.BlockSpec((_BLK, D_FEAT), lambda i: (i, 0)),
    out_shape=jax.ShapeDtypeStruct((N_NODES, D_FEAT), jnp.float32),
)


def kernel(x, edge_index, edge_weight):
    src = edge_index[1].astype(jnp.int32)
    dst = edge_index[0].astype(jnp.int32)
    w = edge_weight.astype(jnp.float32)
    pad = E_PAD - N_EDGES
    src_r = jnp.pad(src, (0, pad)).reshape(NC, NS, S, K)
    dst_r = jnp.pad(dst, (0, pad)).reshape(NC, NS, S, K)
    w_r = jnp.pad(w, (0, pad)).reshape(NC, NS, S, K)
    part = _sc_spmm(x, src_r, dst_r, w_r)
    return _tc_add(part[0], part[1])


# speedup vs baseline: 4.5720x; 4.5720x over previous
import jax
import jax.numpy as jnp
from jax import lax
from jax.experimental import pallas as pl
from jax.experimental.pallas import tpu as pltpu
from jax.experimental.pallas import tpu_sc as plsc

N_NODES = 10000
D_FEAT = 128
N_EDGES = 320000

NC = 2
NS = 16
K = 128
S = (N_EDGES + NC * NS * K - 1) // (NC * NS * K)
E_PAD = NC * NS * S * K
ACC_ROWS = 10112
ROWS_PER_SUB = ACC_ROWS // NS


def _sc_body(x_hbm, src_hbm, dst_hbm, w_hbm, part_hbm,
             src_v, dst_v, w_v, rows_v, acc, sem):
    c = lax.axis_index("c")
    s = lax.axis_index("s")

    pltpu.sync_copy(src_hbm.at[c, s], src_v)
    pltpu.sync_copy(dst_hbm.at[c, s], dst_v)
    pltpu.sync_copy(w_hbm.at[c, s], w_v)

    zero16 = jnp.zeros((16,), jnp.float32)

    def _zrow(i, _):
        for r in range(D_FEAT // 16):
            rows_v[i, pl.ds(r * 16, 16)] = zero16
        return 0

    lax.fori_loop(0, K, _zrow, 0)
    row0 = s * ROWS_PER_SUB
    off = 0
    while off < ROWS_PER_SUB:
        n = min(K, ROWS_PER_SUB - off)
        pltpu.sync_copy(rows_v.at[pl.ds(0, n)], acc.at[pl.ds(row0 + off, n)])
        off += n
    plsc.subcore_barrier()

    def _step(j, _):
        pltpu.async_copy(x_hbm.at[src_v.at[j]], rows_v, sem).wait()

        def _scale(g, _):
            wv16 = w_v[j, pl.ds(g * 16, 16)]
            for l in range(16):
                wl = wv16[l]
                row = g * 16 + l
                for r in range(D_FEAT // 16):
                    sl = pl.ds(r * 16, 16)
                    rows_v[row, sl] = rows_v[row, sl] * wl
            return 0

        lax.fori_loop(0, K // 16, _scale, 0)
        pltpu.sync_copy(rows_v, acc.at[dst_v.at[j]], add=True)
        return 0

    lax.fori_loop(0, S, _step, 0)
    plsc.subcore_barrier()

    pltpu.sync_copy(acc.at[pl.ds(row0, ROWS_PER_SUB)],
                    part_hbm.at[c, pl.ds(row0, ROWS_PER_SUB)])


_sc_spmm = pl.kernel(
    _sc_body,
    out_type=jax.ShapeDtypeStruct((NC, ACC_ROWS, D_FEAT), jnp.float32),
    mesh=plsc.VectorSubcoreMesh(core_axis_name="c", subcore_axis_name="s"),
    scratch_types=[
        pltpu.VMEM((S, K), jnp.int32),
        pltpu.VMEM((S, K), jnp.int32),
        pltpu.VMEM((S, K), jnp.float32),
        pltpu.VMEM((K, D_FEAT), jnp.float32),
        pltpu.VMEM_SHARED((ACC_ROWS, D_FEAT), jnp.float32),
        pltpu.SemaphoreType.DMA,
    ],
)


def _add_body(p_ref, o_ref):
    o_ref[...] = p_ref[0] + p_ref[1]


_BLK = 1000
_tc_add = pl.pallas_call(
    _add_body,
    grid=(N_NODES // _BLK,),
    in_specs=[pl.BlockSpec((NC, _BLK, D_FEAT), lambda i: (0, i, 0))],
    out_specs=pl.BlockSpec((_BLK, D_FEAT), lambda i: (i, 0)),
    out_shape=jax.ShapeDtypeStruct((N_NODES, D_FEAT), jnp.float32),
)


def kernel(x, edge_index, edge_weight):
    src = edge_index[1].astype(jnp.int32)
    dst = edge_index[0].astype(jnp.int32)
    w = edge_weight.astype(jnp.float32)
    pad = E_PAD - N_EDGES
    src_r = jnp.pad(src, (0, pad)).reshape(NC, NS, S, K)
    dst_r = jnp.pad(dst, (0, pad)).reshape(NC, NS, S, K)
    w_r = jnp.pad(w, (0, pad)).reshape(NC, NS, S, K)
    part = _sc_spmm(x, src_r, dst_r, w_r)
    return _tc_add(part)
